# baseline (speedup 1.0000x reference)
import functools

import jax
import jax.numpy as jnp
from jax import lax
from jax.experimental import pallas as pl
from jax.experimental.pallas import tpu as pltpu
from jax.experimental.pallas import tpu_sc as plsc

VOCAB = 151936
D_MODEL = 2048
NUM_TOKENS = 4 * 2048

_NC = 2
_NS = 16
_NW = _NC * _NS

_B_PER_W = NUM_TOKENS // _NW
_CHUNK = 16
_NBUF = 2
_AHEAD = 1
_N_CHUNKS = _B_PER_W // _CHUNK


def _gather_body(ids_hbm, table_hbm, out_hbm, idx_v, rows_v, *sems):
    gsems = sems[:_NBUF]
    ssems = sems[_NBUF:]
    wid = lax.axis_index("s") * _NC + lax.axis_index("c")
    base = wid * _B_PER_W
    pltpu.sync_copy(ids_hbm.at[wid], idx_v)

    def start_gather(i, b):
        pltpu.async_copy(table_hbm.at[idx_v.at[i]], rows_v.at[b], gsems[b])

    def wait_gather(b):
        pltpu.make_async_copy(
            table_hbm.at[pl.ds(0, _CHUNK)], rows_v.at[b], gsems[b]).wait()

    def start_scatter(i, b):
        pltpu.async_copy(
            rows_v.at[b], out_hbm.at[pl.ds(base + i * _CHUNK, _CHUNK)],
            ssems[b])

    def wait_scatter(b):
        pltpu.make_async_copy(
            rows_v.at[b], out_hbm.at[pl.ds(base, _CHUNK)], ssems[b]).wait()

    for j in range(_AHEAD):
        start_gather(j, j % _NBUF)

    def ring(g):
        for b0 in range(_NBUF):
            i = g + b0
            b = b0
            nb = (b0 + _AHEAD) % _NBUF

            @pl.when(i >= _NBUF - _AHEAD)
            def _():
                wait_scatter(nb)

            @pl.when(i + _AHEAD < _N_CHUNKS)
            def _():
                start_gather(i + _AHEAD, nb)

            wait_gather(b)
            start_scatter(i, b)

    pl.loop(0, _N_CHUNKS, step=_NBUF)(ring)

    for j in range(_NBUF - _AHEAD):
        wait_scatter((_N_CHUNKS - 1 - j) % _NBUF)


@functools.partial(
    pl.kernel,
    out_type=jax.ShapeDtypeStruct((NUM_TOKENS, D_MODEL), jnp.float32),
    mesh=plsc.VectorSubcoreMesh(core_axis_name="c", subcore_axis_name="s"),
    scratch_types=[
        pltpu.VMEM((_N_CHUNKS, _CHUNK), jnp.int32),
        pltpu.VMEM((_NBUF, _CHUNK, D_MODEL), jnp.float32),
    ] + [pltpu.SemaphoreType.DMA] * (2 * _NBUF),
)
def _sc_gather(ids_hbm, table_hbm, out_hbm, idx_v, rows_v, *sems):
    _gather_body(ids_hbm, table_hbm, out_hbm, idx_v, rows_v, *sems)


def kernel(input_ids, embed_table):
    ids_flat = jnp.reshape(input_ids, (_NW, _N_CHUNKS, _CHUNK)).astype(
        jnp.int32)
    out = _sc_gather(ids_flat, embed_table)
    return jnp.reshape(out, (*input_ids.shape, D_MODEL))


# speedup vs baseline: 1.0077x; 1.0077x over previous
import functools

import jax
import jax.numpy as jnp
from jax import lax
from jax.experimental import pallas as pl
from jax.experimental.pallas import tpu as pltpu
from jax.experimental.pallas import tpu_sc as plsc

VOCAB = 151936
D_MODEL = 2048
NUM_TOKENS = 4 * 2048

_NC = 2
_NS = 16
_NW = _NC * _NS

_B_PER_W = NUM_TOKENS // _NW
_CHUNK = 8
_NBUF = 4
_AHEAD = 3
_N_CHUNKS = _B_PER_W // _CHUNK


def _gather_body(ids_hbm, table_hbm, out_hbm, idx_v, rows_v, *sems):
    gsems = sems[:_NBUF]
    ssems = sems[_NBUF:]
    wid = lax.axis_index("s") * _NC + lax.axis_index("c")
    base = wid * _B_PER_W
    pltpu.sync_copy(ids_hbm.at[wid], idx_v)

    def start_gather(i, b):
        pltpu.async_copy(table_hbm.at[idx_v.at[i]], rows_v.at[b], gsems[b])

    def wait_gather(b):
        pltpu.make_async_copy(
            table_hbm.at[pl.ds(0, _CHUNK)], rows_v.at[b], gsems[b]).wait()

    def start_scatter(i, b):
        pltpu.async_copy(
            rows_v.at[b], out_hbm.at[pl.ds(base + i * _CHUNK, _CHUNK)],
            ssems[b])

    def wait_scatter(b):
        pltpu.make_async_copy(
            rows_v.at[b], out_hbm.at[pl.ds(base, _CHUNK)], ssems[b]).wait()

    for j in range(_AHEAD):
        start_gather(j, j % _NBUF)

    def ring(g):
        for b0 in range(_NBUF):
            i = g + b0
            b = b0
            nb = (b0 + _AHEAD) % _NBUF

            @pl.when(i >= _NBUF - _AHEAD)
            def _():
                wait_scatter(nb)

            @pl.when(i + _AHEAD < _N_CHUNKS)
            def _():
                start_gather(i + _AHEAD, nb)

            wait_gather(b)
            start_scatter(i, b)

    pl.loop(0, _N_CHUNKS, step=_NBUF)(ring)

    for j in range(_NBUF - _AHEAD):
        wait_scatter((_N_CHUNKS - 1 - j) % _NBUF)


@functools.partial(
    pl.kernel,
    out_type=jax.ShapeDtypeStruct((NUM_TOKENS, D_MODEL), jnp.float32),
    mesh=plsc.VectorSubcoreMesh(core_axis_name="c", subcore_axis_name="s"),
    scratch_types=[
        pltpu.VMEM((_N_CHUNKS, _CHUNK), jnp.int32),
        pltpu.VMEM((_NBUF, _CHUNK, D_MODEL), jnp.float32),
    ] + [pltpu.SemaphoreType.DMA] * (2 * _NBUF),
)
def _sc_gather(ids_hbm, table_hbm, out_hbm, idx_v, rows_v, *sems):
    _gather_body(ids_hbm, table_hbm, out_hbm, idx_v, rows_v, *sems)


def kernel(input_ids, embed_table):
    ids_flat = jnp.reshape(input_ids, (_NW, _N_CHUNKS, _CHUNK)).astype(
        jnp.int32)
    out = _sc_gather(ids_flat, embed_table)
    return jnp.reshape(out, (*input_ids.shape, D_MODEL))
